# Initial kernel scaffold; baseline (speedup 1.0000x reference)
import jax
import jax.numpy as jnp
from jax.experimental import pallas as pl


def kernel(adj_indices, adj_values, feat_indices, feat_values, W1, b1, W2, b2, W3, b3, W_fc, b_fc):
    raise NotImplementedError("write your pallas kernel here")


# speedup vs baseline: 3.5616x; 3.5616x over previous
import functools

import jax
import jax.numpy as jnp
from jax import lax
from jax.experimental import pallas as pl
from jax.experimental.pallas import tpu as pltpu
from jax.experimental.pallas import tpu_sc as plsc

N = 10000
F = 10000
H = 64
WCAT = 3 * H
NUM_CLASSES = 32

K = 128
GROUP = K * 16
ROWS_PER_TILE = 624


def _pad_to(x, total, axis):
    pad = total - x.shape[axis]
    cfg = [(0, 0)] * x.ndim
    cfg[axis] = (0, pad)
    return jnp.pad(x, cfg)


def _make_sc_spmm(nnz_pad, half):
    chunks = nnz_pad // K
    cpt = chunks // 16
    assert chunks % 16 == 0
    mesh = plsc.VectorSubcoreMesh(core_axis_name="c", subcore_axis_name="s")

    @functools.partial(
        pl.kernel,
        mesh=mesh,
        compiler_params=pltpu.CompilerParams(use_tc_tiling_on_sc=False),
        out_type=jax.ShapeDtypeStruct((2, N, half), jnp.float32),
        scratch_types=[
            pltpu.VMEM((K,), jnp.int32),
            pltpu.VMEM((K,), jnp.int32),
            pltpu.VMEM((K, 16), jnp.float32),
            pltpu.VMEM((K, half), jnp.float32),
            pltpu.VMEM_SHARED((N, half), jnp.float32),
            pltpu.SemaphoreType.DMA,
        ],
    )
    def spmm(col_hbm, row_hbm, valx_hbm, taba_hbm, tabb_hbm, z_hbm, out_hbm,
             cidx_v, ridx_v, valx_v, rows_v, acc, sem):
        cid = lax.axis_index("c")
        sid = lax.axis_index("s")

        r0 = sid * ROWS_PER_TILE
        tail0 = 16 * ROWS_PER_TILE
        tail_n = N - tail0
        pltpu.sync_copy(z_hbm.at[pl.ds(r0, ROWS_PER_TILE), :],
                        acc.at[pl.ds(r0, ROWS_PER_TILE), :])

        @pl.when(sid == 15)
        def _zero_tail():
            pltpu.sync_copy(z_hbm.at[pl.ds(tail0, tail_n), :],
                            acc.at[pl.ds(tail0, tail_n), :])

        plsc.subcore_barrier()

        def run(tab_hbm):
            def chunk_body(i, carry):
                base = (sid * cpt + i) * K
                pltpu.sync_copy(col_hbm.at[pl.ds(base, K)], cidx_v)
                pltpu.sync_copy(row_hbm.at[pl.ds(base, K)], ridx_v)
                pltpu.sync_copy(valx_hbm.at[pl.ds(base, K), :], valx_v)
                pltpu.async_copy(tab_hbm.at[cidx_v], rows_v, sem).wait()

                def scale_body(j, c2):
                    v = valx_v[j, :]
                    for g in range(half // 16):
                        sl = pl.ds(g * 16, 16)
                        rows_v[j, sl] = rows_v[j, sl] * v
                    return c2

                lax.fori_loop(0, K, scale_body, 0)
                pltpu.sync_copy(rows_v, acc.at[ridx_v], add=True)
                return carry

            lax.fori_loop(0, cpt, chunk_body, 0)

        @pl.when(cid == 0)
        def _run_a():
            run(taba_hbm)

        @pl.when(cid == 1)
        def _run_b():
            run(tabb_hbm)

        plsc.subcore_barrier()
        pltpu.sync_copy(acc.at[pl.ds(r0, ROWS_PER_TILE), :],
                        out_hbm.at[cid, pl.ds(r0, ROWS_PER_TILE), :])

        @pl.when(sid == 15)
        def _write_tail():
            pltpu.sync_copy(acc.at[pl.ds(tail0, tail_n), :],
                            out_hbm.at[cid, pl.ds(tail0, tail_n), :])

    return spmm


ROW_BLK = 1000


def _tc_combine1_body(p_ref, b_ref, x64_ref, ya_ref, yb_ref):
    x = jnp.concatenate([p_ref[0], p_ref[1]], axis=1)
    x = jnp.maximum(x + b_ref[0][None, :], 0.0)
    x64_ref[...] = x[:, :H]
    ya_ref[...] = x[:, H:2 * H]
    yb_ref[...] = x[:, 2 * H:]


def _tc_combine1(p, bcat):
    grid = N // ROW_BLK
    return pl.pallas_call(
        _tc_combine1_body,
        grid=(grid,),
        in_specs=[
            pl.BlockSpec((2, ROW_BLK, WCAT // 2), lambda i: (0, i, 0)),
            pl.BlockSpec((1, WCAT), lambda i: (0, 0)),
        ],
        out_specs=[
            pl.BlockSpec((ROW_BLK, H), lambda i: (i, 0)),
            pl.BlockSpec((ROW_BLK, H), lambda i: (i, 0)),
            pl.BlockSpec((ROW_BLK, H), lambda i: (i, 0)),
        ],
        out_shape=[
            jax.ShapeDtypeStruct((N, H), jnp.float32),
            jax.ShapeDtypeStruct((N, H), jnp.float32),
            jax.ShapeDtypeStruct((N, H), jnp.float32),
        ],
    )(p, bcat)


def _tc_final_body(x_ref, t_ref, r_ref, wfc_ref, bfc_ref, out_ref):
    a2 = jnp.concatenate([r_ref[0], r_ref[1]], axis=1)
    w = wfc_ref[...]
    logits = jnp.dot(x_ref[...], w[:H], preferred_element_type=jnp.float32)
    logits += jnp.dot(t_ref[...], w[H:2 * H], preferred_element_type=jnp.float32)
    logits += jnp.dot(a2, w[2 * H:], preferred_element_type=jnp.float32)
    logits += bfc_ref[0][None, :]
    m = jnp.max(logits, axis=1, keepdims=True)
    z = logits - m
    lse = jnp.log(jnp.sum(jnp.exp(z), axis=1, keepdims=True))
    out_ref[...] = z - lse


def _tc_final(x64, t64, r, w_fc, b_fc):
    grid = N // ROW_BLK
    return pl.pallas_call(
        _tc_final_body,
        grid=(grid,),
        in_specs=[
            pl.BlockSpec((ROW_BLK, H), lambda i: (i, 0)),
            pl.BlockSpec((ROW_BLK, H), lambda i: (i, 0)),
            pl.BlockSpec((2, ROW_BLK, H // 2), lambda i: (0, i, 0)),
            pl.BlockSpec((WCAT, NUM_CLASSES), lambda i: (0, 0)),
            pl.BlockSpec((1, NUM_CLASSES), lambda i: (0, 0)),
        ],
        out_specs=pl.BlockSpec((ROW_BLK, NUM_CLASSES), lambda i: (i, 0)),
        out_shape=jax.ShapeDtypeStruct((N, NUM_CLASSES), jnp.float32),
    )(x64, t64, r, w_fc, b_fc.reshape(1, NUM_CLASSES))


def _ceil_to(x, m):
    return ((x + m - 1) // m) * m


def kernel(adj_indices, adj_values, feat_indices, feat_values,
           W1, b1, W2, b2, W3, b3, W_fc, b_fc):
    fpad = _ceil_to(feat_indices.shape[1], GROUP)
    apad = _ceil_to(adj_indices.shape[1], GROUP)

    w_a = jnp.concatenate([W1, W2[:, :H // 2]], axis=1)
    w_b = jnp.concatenate([W2[:, H // 2:], W3], axis=1)
    bcat = jnp.concatenate([b1, b2, b3], axis=1)

    f_col = _pad_to(feat_indices[1], fpad, 0)
    f_row = _pad_to(feat_indices[0], fpad, 0)
    f_valx = jnp.broadcast_to(_pad_to(feat_values, fpad, 0)[:, None], (fpad, 16))
    a_col = _pad_to(adj_indices[1], apad, 0)
    a_row = _pad_to(adj_indices[0], apad, 0)
    a_valx = jnp.broadcast_to(_pad_to(adj_values, apad, 0)[:, None], (apad, 16))

    z96 = jnp.zeros((N, WCAT // 2), jnp.float32)
    z64 = jnp.zeros((N, H), jnp.float32)
    z32 = jnp.zeros((N, H // 2), jnp.float32)

    p = _make_sc_spmm(fpad, WCAT // 2)(f_col, f_row, f_valx, w_a, w_b, z96)
    x64, y_a, y_b = _tc_combine1(p, bcat)

    q = _make_sc_spmm(apad, H)(a_col, a_row, a_valx, y_a, y_b, z64)
    t64a, t64b = q[0], q[1]

    r = _make_sc_spmm(apad, H // 2)(
        a_col, a_row, a_valx, t64b[:, :H // 2], t64b[:, H // 2:], z32)

    return _tc_final(x64, t64a, r, W_fc, b_fc)


# speedup vs baseline: 6.6082x; 1.8554x over previous
import functools

import jax
import jax.numpy as jnp
from jax import lax
from jax.experimental import pallas as pl
from jax.experimental.pallas import tpu as pltpu
from jax.experimental.pallas import tpu_sc as plsc

N = 10000
F = 10000
H = 64
WCAT = 3 * H
NUM_CLASSES = 32

K = 128
GROUP = K * 32
ROWS_PER_TILE = 624


def _pad_to(x, total, axis):
    pad = total - x.shape[axis]
    cfg = [(0, 0)] * x.ndim
    cfg[axis] = (0, pad)
    return jnp.pad(x, cfg)


def _make_sc_spmm(nnz_pad, half):
    chunks = nnz_pad // K
    cpt = chunks // 16
    assert chunks % 32 == 0
    mesh = plsc.VectorSubcoreMesh(core_axis_name="c", subcore_axis_name="s")

    @functools.partial(
        pl.kernel,
        mesh=mesh,
        compiler_params=pltpu.CompilerParams(use_tc_tiling_on_sc=False),
        out_type=jax.ShapeDtypeStruct((2, N, half), jnp.float32),
        scratch_types=[
            pltpu.VMEM((2, K), jnp.int32),
            pltpu.VMEM((2, K), jnp.int32),
            pltpu.VMEM((K, 16), jnp.float32),
            pltpu.VMEM((K, 16), jnp.float32),
            pltpu.VMEM((K, half), jnp.float32),
            pltpu.VMEM((K, half), jnp.float32),
            pltpu.VMEM_SHARED((N, half), jnp.float32),
            pltpu.SemaphoreType.DMA,
            pltpu.SemaphoreType.DMA,
            pltpu.SemaphoreType.DMA,
            pltpu.SemaphoreType.DMA,
        ],
    )
    def spmm(pidx_hbm, valx_hbm, taba_hbm, tabb_hbm, z_hbm, out_hbm,
             idx0, idx1, val0, val1, rows0, rows1, acc,
             gsem0, gsem1, ssem0, ssem1):
        cid = lax.axis_index("c")
        sid = lax.axis_index("s")

        r0 = sid * ROWS_PER_TILE
        tail0 = 16 * ROWS_PER_TILE
        tail_n = N - tail0
        pltpu.sync_copy(z_hbm.at[pl.ds(r0, ROWS_PER_TILE), :],
                        acc.at[pl.ds(r0, ROWS_PER_TILE), :])

        @pl.when(sid == 15)
        def _zero_tail():
            pltpu.sync_copy(z_hbm.at[pl.ds(tail0, tail_n), :],
                            acc.at[pl.ds(tail0, tail_n), :])

        plsc.subcore_barrier()
        c0 = sid * cpt

        def run(tab_hbm):
            def scale(rows_v, valx_v):
                @plsc.parallel_loop(0, K, unroll=8)
                def _scale(j):
                    v = valx_v[j, :]
                    for g in range(half // 16):
                        sl = pl.ds(g * 16, 16)
                        rows_v[j, sl] = rows_v[j, sl] * v

            def step(j, idxA, valA, rowsA, gsemA, ssemA,
                     idxB, valB, rowsB, gsemB, ssemB):
                @pl.when(j > 0)
                def _drain_prev_scatter():
                    pltpu.make_async_copy(rowsB, acc.at[idxB.at[1]],
                                          ssemB).wait()

                @pl.when(j + 1 < cpt)
                def _prefetch_next():
                    pltpu.async_copy(pidx_hbm.at[c0 + j + 1], idxB, gsemB)
                    pltpu.async_copy(valx_hbm.at[c0 + j + 1], valB, gsemB)

                pltpu.make_async_copy(tab_hbm.at[idxA.at[0]], rowsA,
                                      gsemA).wait()
                scale(rowsA, valA)

                @pl.when(j + 1 < cpt)
                def _launch_next_gather():
                    pltpu.make_async_copy(pidx_hbm.at[c0 + j + 1], idxB,
                                          gsemB).wait()
                    pltpu.make_async_copy(valx_hbm.at[c0 + j + 1], valB,
                                          gsemB).wait()
                    pltpu.async_copy(tab_hbm.at[idxB.at[0]], rowsB, gsemB)

                pltpu.async_copy(rowsA, acc.at[idxA.at[1]], ssemA, add=True)

            pltpu.async_copy(pidx_hbm.at[c0], idx0, gsem0)
            pltpu.async_copy(valx_hbm.at[c0], val0, gsem0)
            pltpu.make_async_copy(pidx_hbm.at[c0], idx0, gsem0).wait()
            pltpu.make_async_copy(valx_hbm.at[c0], val0, gsem0).wait()
            pltpu.async_copy(tab_hbm.at[idx0.at[0]], rows0, gsem0)

            @pl.loop(0, cpt, step=2)
            def _pair(i):
                step(i, idx0, val0, rows0, gsem0, ssem0,
                     idx1, val1, rows1, gsem1, ssem1)
                step(i + 1, idx1, val1, rows1, gsem1, ssem1,
                     idx0, val0, rows0, gsem0, ssem0)

            pltpu.make_async_copy(rows1, acc.at[idx1.at[1]], ssem1).wait()

        @pl.when(cid == 0)
        def _run_a():
            run(taba_hbm)

        @pl.when(cid == 1)
        def _run_b():
            run(tabb_hbm)

        plsc.subcore_barrier()
        pltpu.sync_copy(acc.at[pl.ds(r0, ROWS_PER_TILE), :],
                        out_hbm.at[cid, pl.ds(r0, ROWS_PER_TILE), :])

        @pl.when(sid == 15)
        def _write_tail():
            pltpu.sync_copy(acc.at[pl.ds(tail0, tail_n), :],
                            out_hbm.at[cid, pl.ds(tail0, tail_n), :])

    return spmm


ROW_BLK = 1000


def _tc_combine1_body(p_ref, b_ref, x64_ref, ya_ref, yb_ref):
    x = jnp.concatenate([p_ref[0], p_ref[1]], axis=1)
    x = jnp.maximum(x + b_ref[0][None, :], 0.0)
    x64_ref[...] = x[:, :H]
    ya_ref[...] = x[:, H:2 * H]
    yb_ref[...] = x[:, 2 * H:]


def _tc_combine1(p, bcat):
    grid = N // ROW_BLK
    return pl.pallas_call(
        _tc_combine1_body,
        grid=(grid,),
        in_specs=[
            pl.BlockSpec((2, ROW_BLK, WCAT // 2), lambda i: (0, i, 0)),
            pl.BlockSpec((1, WCAT), lambda i: (0, 0)),
        ],
        out_specs=[
            pl.BlockSpec((ROW_BLK, H), lambda i: (i, 0)),
            pl.BlockSpec((ROW_BLK, H), lambda i: (i, 0)),
            pl.BlockSpec((ROW_BLK, H), lambda i: (i, 0)),
        ],
        out_shape=[
            jax.ShapeDtypeStruct((N, H), jnp.float32),
            jax.ShapeDtypeStruct((N, H), jnp.float32),
            jax.ShapeDtypeStruct((N, H), jnp.float32),
        ],
    )(p, bcat)


def _tc_final_body(x_ref, t_ref, r_ref, wfc_ref, bfc_ref, out_ref):
    a2 = jnp.concatenate([r_ref[0], r_ref[1]], axis=1)
    w = wfc_ref[...]
    logits = jnp.dot(x_ref[...], w[:H], preferred_element_type=jnp.float32)
    logits += jnp.dot(t_ref[...], w[H:2 * H], preferred_element_type=jnp.float32)
    logits += jnp.dot(a2, w[2 * H:], preferred_element_type=jnp.float32)
    logits += bfc_ref[0][None, :]
    m = jnp.max(logits, axis=1, keepdims=True)
    z = logits - m
    lse = jnp.log(jnp.sum(jnp.exp(z), axis=1, keepdims=True))
    out_ref[...] = z - lse


def _tc_final(x64, t64, r, w_fc, b_fc):
    grid = N // ROW_BLK
    return pl.pallas_call(
        _tc_final_body,
        grid=(grid,),
        in_specs=[
            pl.BlockSpec((ROW_BLK, H), lambda i: (i, 0)),
            pl.BlockSpec((ROW_BLK, H), lambda i: (i, 0)),
            pl.BlockSpec((2, ROW_BLK, H // 2), lambda i: (0, i, 0)),
            pl.BlockSpec((WCAT, NUM_CLASSES), lambda i: (0, 0)),
            pl.BlockSpec((1, NUM_CLASSES), lambda i: (0, 0)),
        ],
        out_specs=pl.BlockSpec((ROW_BLK, NUM_CLASSES), lambda i: (i, 0)),
        out_shape=jax.ShapeDtypeStruct((N, NUM_CLASSES), jnp.float32),
    )(x64, t64, r, w_fc, b_fc.reshape(1, NUM_CLASSES))


def _ceil_to(x, m):
    return ((x + m - 1) // m) * m


def kernel(adj_indices, adj_values, feat_indices, feat_values,
           W1, b1, W2, b2, W3, b3, W_fc, b_fc):
    fpad = _ceil_to(feat_indices.shape[1], GROUP)
    apad = _ceil_to(adj_indices.shape[1], GROUP)

    w_a = jnp.concatenate([W1, W2[:, :H // 2]], axis=1)
    w_b = jnp.concatenate([W2[:, H // 2:], W3], axis=1)
    bcat = jnp.concatenate([b1, b2, b3], axis=1)

    def pack(indices, values, pad):
        col = _pad_to(indices[1], pad, 0).reshape(pad // K, 1, K)
        row = _pad_to(indices[0], pad, 0).reshape(pad // K, 1, K)
        pidx = jnp.concatenate([col, row], axis=1)
        valx = jnp.broadcast_to(
            _pad_to(values, pad, 0).reshape(pad // K, K)[:, :, None],
            (pad // K, K, 16))
        return pidx, valx

    f_pidx, f_valx = pack(feat_indices, feat_values, fpad)
    a_pidx, a_valx = pack(adj_indices, adj_values, apad)

    z96 = jnp.zeros((N, WCAT // 2), jnp.float32)
    z64 = jnp.zeros((N, H), jnp.float32)
    z32 = jnp.zeros((N, H // 2), jnp.float32)

    p = _make_sc_spmm(fpad, WCAT // 2)(f_pidx, f_valx, w_a, w_b, z96)
    x64, y_a, y_b = _tc_combine1(p, bcat)

    q = _make_sc_spmm(apad, H)(a_pidx, a_valx, y_a, y_b, z64)
    t64a, t64b = q[0], q[1]

    r = _make_sc_spmm(apad, H // 2)(
        a_pidx, a_valx, t64b[:, :H // 2], t64b[:, H // 2:], z32)

    return _tc_final(x64, t64a, r, W_fc, b_fc)
